# Initial kernel scaffold; baseline (speedup 1.0000x reference)
import jax
import jax.numpy as jnp
from jax.experimental import pallas as pl


def kernel(x, edge_index, W1a, b1a, W1b, b1b, g1, be1, W2a, b2a, W2b, b2b, g2, be2, Wfc, bfc):
    raise NotImplementedError("write your pallas kernel here")


# speedup vs baseline: 3.9778x; 3.9778x over previous
import functools

import jax
import jax.numpy as jnp
from jax import lax
from jax.experimental import pallas as pl
from jax.experimental.pallas import tpu as pltpu
from jax.experimental.pallas import tpu_sc as plsc

N, E, DIN, H, C = 10000, 320000, 128, 256, 64
NP = 10240

_info = plsc.get_sparse_core_info()
NC, NS = _info.num_cores, _info.num_subcores
K = 80

_f32 = jnp.float32


def _make_agg(table_rows, feature_split):
    mesh = plsc.VectorSubcoreMesh(core_axis_name="c", subcore_axis_name="s")
    if feature_split:
        ew = E // NS
    else:
        ew = E // (NS * NC)
    n_chunks = ew // K
    zr = NP // NS

    @functools.partial(
        pl.kernel,
        mesh=mesh,
        out_type=jax.ShapeDtypeStruct((2 * NP, 128), _f32),
        scratch_types=[
            pltpu.VMEM((K,), jnp.int32),
            pltpu.VMEM((K,), jnp.int32),
            pltpu.VMEM((K, 128), _f32),
            pltpu.VMEM_SHARED((NP, 128), _f32),
            pltpu.SemaphoreType.DMA,
        ],
    )
    def agg(src_hbm, dst_hbm, table_hbm, zeros_hbm, out_hbm,
            idx_s, idx_d, rows, acc, sem):
        c = lax.axis_index("c")
        s = lax.axis_index("s")
        pltpu.sync_copy(zeros_hbm.at[pl.ds(s * zr, zr)], acc.at[pl.ds(s * zr, zr)])
        plsc.subcore_barrier()
        if feature_split:
            base_e = s * ew
            off = c * N
        else:
            base_e = (s * NC + c) * ew
            off = None
        def chunk(i, carry):
            e0 = base_e + i * K
            pltpu.sync_copy(src_hbm.at[pl.ds(e0, K)], idx_s)
            pltpu.sync_copy(dst_hbm.at[pl.ds(e0, K)], idx_d)
            if off is not None:
                for j in range(K // 16):
                    sl = pl.ds(j * 16, 16)
                    idx_s[sl] = idx_s[sl] + off
            pltpu.async_copy(table_hbm.at[idx_s], rows, sem).wait()
            pltpu.sync_copy(rows, acc.at[idx_d], add=True)
            return carry
        lax.fori_loop(0, n_chunks, chunk, 0)
        plsc.subcore_barrier()
        pltpu.sync_copy(acc.at[pl.ds(s * zr, zr)],
                        out_hbm.at[pl.ds(c * NP + s * zr, zr)])

    return agg


_agg_l1 = _make_agg(N, feature_split=False)
_agg_l2 = _make_agg(2 * N, feature_split=True)


_BN = 1000
_NB = N // _BN


def _conv1_mlp(x_ref, agg_ref, wa_ref, ba_ref, wb_ref, bb_ref,
               u_ref, s_ref, q_ref):
    h = x_ref[...] + agg_ref[0] + agg_ref[1]
    t = jnp.maximum(
        jnp.dot(h, wa_ref[...], preferred_element_type=_f32) + ba_ref[...], 0.0)
    u = jnp.dot(t, wb_ref[...], preferred_element_type=_f32) + bb_ref[...]
    u_ref[...] = u
    @pl.when(pl.program_id(0) == 0)
    def _():
        s_ref[...] = jnp.zeros_like(s_ref)
        q_ref[...] = jnp.zeros_like(q_ref)
    s_ref[...] += jnp.sum(u, axis=0, keepdims=True)
    q_ref[...] += jnp.sum(u * u, axis=0, keepdims=True)


def _conv2_mlp(hb_ref, agg_ref, wa_ref, ba_ref, wb_ref, bb_ref,
               u_ref, s_ref, q_ref):
    h = jnp.concatenate(
        [hb_ref[0] + agg_ref[0], hb_ref[1] + agg_ref[1]], axis=1)
    t = jnp.maximum(
        jnp.dot(h, wa_ref[...], preferred_element_type=_f32) + ba_ref[...], 0.0)
    u = jnp.dot(t, wb_ref[...], preferred_element_type=_f32) + bb_ref[...]
    u_ref[...] = u
    @pl.when(pl.program_id(0) == 0)
    def _():
        s_ref[...] = jnp.zeros_like(s_ref)
        q_ref[...] = jnp.zeros_like(q_ref)
    s_ref[...] += jnp.sum(u, axis=0, keepdims=True)
    q_ref[...] += jnp.sum(u * u, axis=0, keepdims=True)


def _bn_relu_split(u_ref, s_ref, q_ref, g_ref, be_ref, o_ref):
    mu = s_ref[...] * (1.0 / N)
    var = q_ref[...] * (1.0 / N) - mu * mu
    inv = lax.rsqrt(var + 1e-5)
    hb = jnp.maximum((u_ref[...] - mu) * (inv * g_ref[...]) + be_ref[...], 0.0)
    o_ref[0] = hb[:, :128]
    o_ref[1] = hb[:, 128:]


def _bn_fc_lsm(v_ref, s_ref, q_ref, g_ref, be_ref, wfc_ref, bfc_ref, o_ref):
    mu = s_ref[...] * (1.0 / N)
    var = q_ref[...] * (1.0 / N) - mu * mu
    inv = lax.rsqrt(var + 1e-5)
    hc = jnp.maximum((v_ref[...] - mu) * (inv * g_ref[...]) + be_ref[...], 0.0)
    logits = jnp.dot(hc, wfc_ref[...], preferred_element_type=_f32) + bfc_ref[...]
    m = jnp.max(logits, axis=1, keepdims=True)
    lse = jnp.log(jnp.sum(jnp.exp(logits - m), axis=1, keepdims=True))
    o_ref[...] = logits - m - lse


def _row_spec(w):
    return pl.BlockSpec((_BN, w), lambda i: (i, 0))


def _split_spec():
    return pl.BlockSpec((2, _BN, 128), lambda i: (0, i, 0))


def _full_spec(shape):
    return pl.BlockSpec(shape, lambda i: tuple(0 for _ in shape))


def _conv_call(body, h_in, hspec, agg, wa, ba, wb, bb):
    return pl.pallas_call(
        body,
        grid=(_NB,),
        in_specs=[
            hspec,
            _split_spec(),
            _full_spec(wa.shape),
            _full_spec((1, H)),
            _full_spec(wb.shape),
            _full_spec((1, H)),
        ],
        out_specs=[
            _row_spec(H),
            _full_spec((1, H)),
            _full_spec((1, H)),
        ],
        out_shape=[
            jax.ShapeDtypeStruct((N, H), _f32),
            jax.ShapeDtypeStruct((1, H), _f32),
            jax.ShapeDtypeStruct((1, H), _f32),
        ],
    )(h_in, agg, wa, ba.reshape(1, H), wb, bb.reshape(1, H))


def kernel(x, edge_index, W1a, b1a, W1b, b1b, g1, be1,
           W2a, b2a, W2b, b2b, g2, be2, Wfc, bfc):
    src = edge_index[0]
    dst = edge_index[1]
    zeros = jnp.zeros((NP, 128), _f32)

    agg1 = _agg_l1(src, dst, x, zeros).reshape(2, NP, 128)[:, :N]
    u1, s1, q1 = _conv_call(_conv1_mlp, x, _row_spec(DIN), agg1,
                            W1a, b1a, W1b, b1b)
    hb1 = pl.pallas_call(
        _bn_relu_split,
        grid=(_NB,),
        in_specs=[_row_spec(H), _full_spec((1, H)), _full_spec((1, H)),
                  _full_spec((1, H)), _full_spec((1, H))],
        out_specs=_split_spec(),
        out_shape=jax.ShapeDtypeStruct((2, N, 128), _f32),
    )(u1, s1, q1, g1.reshape(1, H), be1.reshape(1, H))

    agg2 = _agg_l2(src, dst, hb1.reshape(2 * N, 128),
                   zeros).reshape(2, NP, 128)[:, :N]
    u2, s2, q2 = _conv_call(_conv2_mlp, hb1, _split_spec(), agg2,
                            W2a, b2a, W2b, b2b)
    out = pl.pallas_call(
        _bn_fc_lsm,
        grid=(_NB,),
        in_specs=[_row_spec(H), _full_spec((1, H)), _full_spec((1, H)),
                  _full_spec((1, H)), _full_spec((1, H)),
                  _full_spec((H, C)), _full_spec((1, C))],
        out_specs=_row_spec(C),
        out_shape=jax.ShapeDtypeStruct((N, C), _f32),
    )(u2, s2, q2, g2.reshape(1, H), be2.reshape(1, H),
      Wfc, bfc.reshape(1, C))
    return out


# speedup vs baseline: 10.3607x; 2.6047x over previous
import functools

import jax
import jax.numpy as jnp
from jax import lax
from jax.experimental import pallas as pl
from jax.experimental.pallas import tpu as pltpu
from jax.experimental.pallas import tpu_sc as plsc

N, E, DIN, H, C = 10000, 320000, 128, 256, 64
NP = 10240

_info = plsc.get_sparse_core_info()
NC, NS = _info.num_cores, _info.num_subcores
K = 80
NBUF = 4

_f32 = jnp.float32


def _make_agg(feature_split):
    mesh = plsc.VectorSubcoreMesh(core_axis_name="c", subcore_axis_name="s")
    if feature_split:
        n = E // (NS * K)
        ew = E // NS
    else:
        n = E // (NS * NC * K)
        ew = E // (NS * NC)
    zr = NP // NS

    NI = 2 * NBUF
    LI = NBUF + 1
    LG = 2

    @functools.partial(
        pl.kernel,
        mesh=mesh,
        out_type=jax.ShapeDtypeStruct((2 * NP, 128), _f32),
        scratch_types=(
            [pltpu.VMEM((K,), jnp.int32) for _ in range(2 * NI)]
            + [pltpu.VMEM((K, 128), _f32) for _ in range(NBUF)]
            + [pltpu.SemaphoreType.DMA for _ in range(NI + 2 * NBUF)]
            + [pltpu.VMEM_SHARED((NP, 128), _f32)]
        ),
    )
    def agg(src_hbm, dst_hbm, table_hbm, zeros_hbm, out_hbm, *refs):
        isrc = list(refs[0:NI])
        idst = list(refs[NI:2 * NI])
        rows = list(refs[2 * NI:2 * NI + NBUF])
        isem = list(refs[2 * NI + NBUF:3 * NI + NBUF])
        gsem = list(refs[3 * NI + NBUF:3 * NI + 2 * NBUF])
        ssem = list(refs[3 * NI + 2 * NBUF:3 * NI + 3 * NBUF])
        acc = refs[3 * NI + 3 * NBUF]
        c = lax.axis_index("c")
        s = lax.axis_index("s")
        if feature_split:
            src_base = c * E + s * ew
            dst_base = s * ew
        else:
            src_base = (s * NC + c) * ew
            dst_base = src_base

        def iload(j, h):
            pltpu.async_copy(src_hbm.at[pl.ds(src_base + j * K, K)],
                             isrc[h], isem[h])
            pltpu.async_copy(dst_hbm.at[pl.ds(dst_base + j * K, K)],
                             idst[h], isem[h])

        def iload_wait(h):
            pltpu.make_async_copy(src_hbm.at[pl.ds(0, K)], isrc[h],
                                  isem[h]).wait()
            pltpu.make_async_copy(dst_hbm.at[pl.ds(0, K)], idst[h],
                                  isem[h]).wait()

        def gather(h, b):
            pltpu.async_copy(table_hbm.at[isrc[h]], rows[b], gsem[b])

        def gather_wait(h, b):
            pltpu.make_async_copy(table_hbm.at[isrc[h]], rows[b],
                                  gsem[b]).wait()

        def scatter(h, b):
            pltpu.async_copy(rows[b], acc.at[idst[h]], ssem[b], add=True)

        def scatter_wait(h, b):
            pltpu.make_async_copy(rows[b], acc.at[idst[h]], ssem[b]).wait()

        for j in range(LI):
            iload(j, j)
        pltpu.sync_copy(zeros_hbm.at[pl.ds(s * zr, zr)],
                        acc.at[pl.ds(s * zr, zr)])
        for j in range(LG):
            iload_wait(j)
            gather(j, j)
        plsc.subcore_barrier()

        def body(i, h):
            b = h % NBUF
            jg = i + LG
            @pl.when(jnp.logical_and(i >= LG, jg < n))
            def _():
                scatter_wait((h - LG) % NI, (b + LG) % NBUF)
            ji = i + LI
            @pl.when(ji < n)
            def _():
                iload(ji, (h + LI) % NI)
            @pl.when(jg < n)
            def _():
                iload_wait((h + LG) % NI)
                gather((h + LG) % NI, (b + LG) % NBUF)
            gather_wait(h, b)
            scatter(h, b)

        def group(g, carry):
            for u in range(NI):
                body(g * NI + u, u)
            return carry
        lax.fori_loop(0, n // NI, group, 0)
        for i in range(NI * (n // NI), n):
            body(jnp.int32(i), i % NI)
        for k in range(NBUF):
            scatter_wait((n - NBUF + k) % NI, (n - NBUF + k) % NBUF)
        plsc.subcore_barrier()
        pltpu.sync_copy(acc.at[pl.ds(s * zr, zr)],
                        out_hbm.at[pl.ds(c * NP + s * zr, zr)])

    return agg


_agg_l1 = _make_agg(feature_split=False)
_agg_l2 = _make_agg(feature_split=True)


_BN = 1000
_NB = N // _BN


def _conv1_mlp(x_ref, agg_ref, wa_ref, ba_ref, wb_ref, bb_ref,
               u_ref, s_ref, q_ref):
    h = x_ref[...] + agg_ref[0] + agg_ref[1]
    t = jnp.maximum(
        jnp.dot(h, wa_ref[...], preferred_element_type=_f32) + ba_ref[...], 0.0)
    u = jnp.dot(t, wb_ref[...], preferred_element_type=_f32) + bb_ref[...]
    u_ref[...] = u
    @pl.when(pl.program_id(0) == 0)
    def _():
        s_ref[...] = jnp.zeros_like(s_ref)
        q_ref[...] = jnp.zeros_like(q_ref)
    s_ref[...] += jnp.sum(u, axis=0, keepdims=True)
    q_ref[...] += jnp.sum(u * u, axis=0, keepdims=True)


def _conv2_mlp(hb_ref, agg_ref, wa_ref, ba_ref, wb_ref, bb_ref,
               u_ref, s_ref, q_ref):
    h = jnp.concatenate(
        [hb_ref[0] + agg_ref[0], hb_ref[1] + agg_ref[1]], axis=1)
    t = jnp.maximum(
        jnp.dot(h, wa_ref[...], preferred_element_type=_f32) + ba_ref[...], 0.0)
    u = jnp.dot(t, wb_ref[...], preferred_element_type=_f32) + bb_ref[...]
    u_ref[...] = u
    @pl.when(pl.program_id(0) == 0)
    def _():
        s_ref[...] = jnp.zeros_like(s_ref)
        q_ref[...] = jnp.zeros_like(q_ref)
    s_ref[...] += jnp.sum(u, axis=0, keepdims=True)
    q_ref[...] += jnp.sum(u * u, axis=0, keepdims=True)


def _bn_relu_split(u_ref, s_ref, q_ref, g_ref, be_ref, o_ref):
    mu = s_ref[...] * (1.0 / N)
    var = q_ref[...] * (1.0 / N) - mu * mu
    inv = lax.rsqrt(var + 1e-5)
    hb = jnp.maximum((u_ref[...] - mu) * (inv * g_ref[...]) + be_ref[...], 0.0)
    o_ref[0] = hb[:, :128]
    o_ref[1] = hb[:, 128:]


def _bn_fc_lsm(v_ref, s_ref, q_ref, g_ref, be_ref, wfc_ref, bfc_ref, o_ref):
    mu = s_ref[...] * (1.0 / N)
    var = q_ref[...] * (1.0 / N) - mu * mu
    inv = lax.rsqrt(var + 1e-5)
    hc = jnp.maximum((v_ref[...] - mu) * (inv * g_ref[...]) + be_ref[...], 0.0)
    logits = jnp.dot(hc, wfc_ref[...], preferred_element_type=_f32) + bfc_ref[...]
    m = jnp.max(logits, axis=1, keepdims=True)
    lse = jnp.log(jnp.sum(jnp.exp(logits - m), axis=1, keepdims=True))
    o_ref[...] = logits - m - lse


def _row_spec(w):
    return pl.BlockSpec((_BN, w), lambda i: (i, 0))


def _split_spec():
    return pl.BlockSpec((2, _BN, 128), lambda i: (0, i, 0))


def _full_spec(shape):
    return pl.BlockSpec(shape, lambda i: tuple(0 for _ in shape))


def _conv_call(body, h_in, hspec, agg, wa, ba, wb, bb):
    return pl.pallas_call(
        body,
        grid=(_NB,),
        in_specs=[
            hspec,
            _split_spec(),
            _full_spec(wa.shape),
            _full_spec((1, H)),
            _full_spec(wb.shape),
            _full_spec((1, H)),
        ],
        out_specs=[
            _row_spec(H),
            _full_spec((1, H)),
            _full_spec((1, H)),
        ],
        out_shape=[
            jax.ShapeDtypeStruct((N, H), _f32),
            jax.ShapeDtypeStruct((1, H), _f32),
            jax.ShapeDtypeStruct((1, H), _f32),
        ],
    )(h_in, agg, wa, ba.reshape(1, H), wb, bb.reshape(1, H))


def kernel(x, edge_index, W1a, b1a, W1b, b1b, g1, be1,
           W2a, b2a, W2b, b2b, g2, be2, Wfc, bfc):
    src = edge_index[0]
    dst = edge_index[1]
    src_both = jnp.concatenate([src, src + N], axis=0)
    zeros = jnp.zeros((NP, 128), _f32)

    agg1 = _agg_l1(src, dst, x, zeros).reshape(2, NP, 128)[:, :N]
    u1, s1, q1 = _conv_call(_conv1_mlp, x, _row_spec(DIN), agg1,
                            W1a, b1a, W1b, b1b)
    hb1 = pl.pallas_call(
        _bn_relu_split,
        grid=(_NB,),
        in_specs=[_row_spec(H), _full_spec((1, H)), _full_spec((1, H)),
                  _full_spec((1, H)), _full_spec((1, H))],
        out_specs=_split_spec(),
        out_shape=jax.ShapeDtypeStruct((2, N, 128), _f32),
    )(u1, s1, q1, g1.reshape(1, H), be1.reshape(1, H))

    agg2 = _agg_l2(src_both, dst, hb1.reshape(2 * N, 128),
                   zeros).reshape(2, NP, 128)[:, :N]
    u2, s2, q2 = _conv_call(_conv2_mlp, hb1, _split_spec(), agg2,
                            W2a, b2a, W2b, b2b)
    out = pl.pallas_call(
        _bn_fc_lsm,
        grid=(_NB,),
        in_specs=[_row_spec(H), _full_spec((1, H)), _full_spec((1, H)),
                  _full_spec((1, H)), _full_spec((1, H)),
                  _full_spec((H, C)), _full_spec((1, C))],
        out_specs=_row_spec(C),
        out_shape=jax.ShapeDtypeStruct((N, C), _f32),
    )(u2, s2, q2, g2.reshape(1, H), be2.reshape(1, H),
      Wfc, bfc.reshape(1, C))
    return out


# speedup vs baseline: 10.9337x; 1.0553x over previous
import functools

import jax
import jax.numpy as jnp
from jax import lax
from jax.experimental import pallas as pl
from jax.experimental.pallas import tpu as pltpu
from jax.experimental.pallas import tpu_sc as plsc

N, E, DIN, H, C = 10000, 320000, 128, 256, 64
NP = 10240

_info = plsc.get_sparse_core_info()
NC, NS = _info.num_cores, _info.num_subcores
K = 80
NBUF = 4

_f32 = jnp.float32


def _make_agg(feature_split):
    mesh = plsc.VectorSubcoreMesh(core_axis_name="c", subcore_axis_name="s")
    if feature_split:
        n = E // (NS * K)
        ew = E // NS
    else:
        n = E // (NS * NC * K)
        ew = E // (NS * NC)
    zr = NP // NS

    NI = 2 * NBUF
    LI = NBUF + 1
    LG = 2

    @functools.partial(
        pl.kernel,
        mesh=mesh,
        out_type=jax.ShapeDtypeStruct((2 * NP, 128), _f32),
        scratch_types=(
            [pltpu.VMEM((K,), jnp.int32) for _ in range(2 * NI)]
            + [pltpu.VMEM((K, 128), _f32) for _ in range(NBUF)]
            + [pltpu.SemaphoreType.DMA for _ in range(NI + 2 * NBUF)]
            + [pltpu.VMEM_SHARED((NP, 128), _f32)]
        ),
    )
    def agg(src_hbm, dst_hbm, table_hbm, out_hbm, *refs):
        isrc = list(refs[0:NI])
        idst = list(refs[NI:2 * NI])
        rows = list(refs[2 * NI:2 * NI + NBUF])
        isem = list(refs[2 * NI + NBUF:3 * NI + NBUF])
        gsem = list(refs[3 * NI + NBUF:3 * NI + 2 * NBUF])
        ssem = list(refs[3 * NI + 2 * NBUF:3 * NI + 3 * NBUF])
        acc = refs[3 * NI + 3 * NBUF]
        c = lax.axis_index("c")
        s = lax.axis_index("s")
        if feature_split:
            src_base = c * E + s * ew
            dst_base = s * ew
        else:
            src_base = (s * NC + c) * ew
            dst_base = src_base

        def iload(j, h):
            pltpu.async_copy(src_hbm.at[pl.ds(src_base + j * K, K)],
                             isrc[h], isem[h])
            pltpu.async_copy(dst_hbm.at[pl.ds(dst_base + j * K, K)],
                             idst[h], isem[h])

        def iload_wait(h):
            pltpu.make_async_copy(src_hbm.at[pl.ds(0, K)], isrc[h],
                                  isem[h]).wait()
            pltpu.make_async_copy(dst_hbm.at[pl.ds(0, K)], idst[h],
                                  isem[h]).wait()

        def gather(h, b):
            pltpu.async_copy(table_hbm.at[isrc[h]], rows[b], gsem[b])

        def gather_wait(h, b):
            pltpu.make_async_copy(table_hbm.at[isrc[h]], rows[b],
                                  gsem[b]).wait()

        def scatter(h, b):
            pltpu.async_copy(rows[b], acc.at[idst[h]], ssem[b], add=True)

        def scatter_wait(h, b):
            pltpu.make_async_copy(rows[b], acc.at[idst[h]], ssem[b]).wait()

        for j in range(LI):
            iload(j, j)
        def zrow(r, carry):
            for jj in range(128 // 16):
                rows[0][r, pl.ds(jj * 16, 16)] = jnp.zeros((16,), _f32)
            return carry
        lax.fori_loop(0, K, zrow, 0)
        for t in range(zr // K):
            pltpu.sync_copy(rows[0], acc.at[pl.ds(s * zr + t * K, K)])
        for j in range(LG):
            iload_wait(j)
            gather(j, j)
        plsc.subcore_barrier()

        def body(i, h):
            b = h % NBUF
            jg = i + LG
            @pl.when(jnp.logical_and(i >= LG, jg < n))
            def _():
                scatter_wait((h - LG) % NI, (b + LG) % NBUF)
            ji = i + LI
            @pl.when(ji < n)
            def _():
                iload(ji, (h + LI) % NI)
            @pl.when(jg < n)
            def _():
                iload_wait((h + LG) % NI)
                gather((h + LG) % NI, (b + LG) % NBUF)
            gather_wait(h, b)
            scatter(h, b)

        def group(g, carry):
            for u in range(NI):
                body(g * NI + u, u)
            return carry
        lax.fori_loop(0, n // NI, group, 0)
        for i in range(NI * (n // NI), n):
            body(jnp.int32(i), i % NI)
        for k in range(NBUF):
            scatter_wait((n - NBUF + k) % NI, (n - NBUF + k) % NBUF)
        plsc.subcore_barrier()
        pltpu.sync_copy(acc.at[pl.ds(s * zr, zr)],
                        out_hbm.at[pl.ds(c * NP + s * zr, zr)])

    return agg


_agg_l1 = _make_agg(feature_split=False)
_agg_l2 = _make_agg(feature_split=True)


_BN = 1000
_NB = N // _BN


def _bn_apply(u, s_scr, q_scr, g_ref, be_ref):
    mu = s_scr[...] * (1.0 / N)
    var = q_scr[...] * (1.0 / N) - mu * mu
    inv = lax.rsqrt(var + 1e-5)
    return jnp.maximum((u - mu) * (inv * g_ref[...]) + be_ref[...], 0.0)


def _mlp_phase(h, wa_ref, ba_ref, wb_ref, bb_ref, i, u_scr, s_scr, q_scr):
    t = jnp.maximum(
        jnp.dot(h, wa_ref[...], preferred_element_type=_f32) + ba_ref[...], 0.0)
    u = jnp.dot(t, wb_ref[...], preferred_element_type=_f32) + bb_ref[...]
    u_scr[pl.ds(i * _BN, _BN), :] = u
    @pl.when(i == 0)
    def _():
        s_scr[...] = jnp.zeros_like(s_scr)
        q_scr[...] = jnp.zeros_like(q_scr)
    s_scr[...] += jnp.sum(u, axis=0, keepdims=True)
    q_scr[...] += jnp.sum(u * u, axis=0, keepdims=True)


def _layer1(x_ref, agg_ref, wa_ref, ba_ref, wb_ref, bb_ref, g_ref, be_ref,
            o_ref, u_scr, s_scr, q_scr):
    p, i = pl.program_id(0), pl.program_id(1)
    @pl.when(p == 0)
    def _():
        h = x_ref[...] + agg_ref[0] + agg_ref[1]
        _mlp_phase(h, wa_ref, ba_ref, wb_ref, bb_ref, i, u_scr, s_scr, q_scr)
    @pl.when(p == 1)
    def _():
        hb = _bn_apply(u_scr[pl.ds(i * _BN, _BN), :], s_scr, q_scr,
                       g_ref, be_ref)
        o_ref[0] = hb[:, :128]
        o_ref[1] = hb[:, 128:]


def _layer2(hb_ref, agg_ref, wa_ref, ba_ref, wb_ref, bb_ref, g_ref, be_ref,
            wfc_ref, bfc_ref, o_ref, u_scr, s_scr, q_scr):
    p, i = pl.program_id(0), pl.program_id(1)
    @pl.when(p == 0)
    def _():
        h = jnp.concatenate(
            [hb_ref[0] + agg_ref[0], hb_ref[1] + agg_ref[1]], axis=1)
        _mlp_phase(h, wa_ref, ba_ref, wb_ref, bb_ref, i, u_scr, s_scr, q_scr)
    @pl.when(p == 1)
    def _():
        hc = _bn_apply(u_scr[pl.ds(i * _BN, _BN), :], s_scr, q_scr,
                       g_ref, be_ref)
        logits = (jnp.dot(hc, wfc_ref[...], preferred_element_type=_f32)
                  + bfc_ref[...])
        m = jnp.max(logits, axis=1, keepdims=True)
        lse = jnp.log(jnp.sum(jnp.exp(logits - m), axis=1, keepdims=True))
        o_ref[...] = logits - m - lse


def _p0_row_spec(w):
    return pl.BlockSpec((_BN, w), lambda p, i: (i * (1 - p), 0))


def _p0_split_spec():
    return pl.BlockSpec((2, _BN, 128), lambda p, i: (0, i * (1 - p), 0))


def _full_spec(shape):
    return pl.BlockSpec(shape, lambda p, i: tuple(0 for _ in shape))


_SCRATCH = [
    pltpu.VMEM((N, H), _f32),
    pltpu.VMEM((1, H), _f32),
    pltpu.VMEM((1, H), _f32),
]


def kernel(x, edge_index, W1a, b1a, W1b, b1b, g1, be1,
           W2a, b2a, W2b, b2b, g2, be2, Wfc, bfc):
    src = edge_index[0]
    dst = edge_index[1]
    src_both = jnp.concatenate([src, src + N], axis=0)

    agg1 = _agg_l1(src, dst, x).reshape(2, NP, 128)[:, :N]
    hb1 = pl.pallas_call(
        _layer1,
        grid=(2, _NB),
        in_specs=[_p0_row_spec(DIN), _p0_split_spec(),
                  _full_spec((DIN, H)), _full_spec((1, H)),
                  _full_spec((H, H)), _full_spec((1, H)),
                  _full_spec((1, H)), _full_spec((1, H))],
        out_specs=pl.BlockSpec((2, _BN, 128), lambda p, i: (0, i * p, 0)),
        out_shape=jax.ShapeDtypeStruct((2, N, 128), _f32),
        scratch_shapes=_SCRATCH,
    )(x, agg1, W1a, b1a.reshape(1, H), W1b, b1b.reshape(1, H),
      g1.reshape(1, H), be1.reshape(1, H))

    agg2 = _agg_l2(src_both, dst, hb1.reshape(2 * N, 128)
                   ).reshape(2, NP, 128)[:, :N]
    out = pl.pallas_call(
        _layer2,
        grid=(2, _NB),
        in_specs=[_p0_split_spec(), _p0_split_spec(),
                  _full_spec((H, H)), _full_spec((1, H)),
                  _full_spec((H, H)), _full_spec((1, H)),
                  _full_spec((1, H)), _full_spec((1, H)),
                  _full_spec((H, C)), _full_spec((1, C))],
        out_specs=pl.BlockSpec((_BN, C), lambda p, i: (i * p, 0)),
        out_shape=jax.ShapeDtypeStruct((N, C), _f32),
        scratch_shapes=_SCRATCH,
    )(hb1, agg2, W2a, b2a.reshape(1, H), W2b, b2b.reshape(1, H),
      g2.reshape(1, H), be2.reshape(1, H), Wfc, bfc.reshape(1, C))
    return out


# speedup vs baseline: 11.2428x; 1.0283x over previous
import functools

import jax
import jax.numpy as jnp
from jax import lax
from jax.experimental import pallas as pl
from jax.experimental.pallas import tpu as pltpu
from jax.experimental.pallas import tpu_sc as plsc

N, E, DIN, H, C = 10000, 320000, 128, 256, 64
NP = 10240

_info = plsc.get_sparse_core_info()
NC, NS = _info.num_cores, _info.num_subcores
K = 80
NBUF = 4

_f32 = jnp.float32


def _make_agg(feature_split):
    mesh = plsc.VectorSubcoreMesh(core_axis_name="c", subcore_axis_name="s")
    if feature_split:
        n = E // (NS * K)
        ew = E // NS
    else:
        n = E // (NS * NC * K)
        ew = E // (NS * NC)
    zr = NP // NS

    NI = 2 * NBUF
    LI = NBUF + 1
    LG = 2

    @functools.partial(
        pl.kernel,
        mesh=mesh,
        out_type=jax.ShapeDtypeStruct((2 * NP, 128), _f32),
        scratch_types=(
            [pltpu.VMEM((K,), jnp.int32) for _ in range(2 * NI)]
            + [pltpu.VMEM((K, 128), _f32) for _ in range(NBUF)]
            + [pltpu.SemaphoreType.DMA for _ in range(NI + 2 * NBUF)]
            + [pltpu.VMEM_SHARED((NP, 128), _f32)]
        ),
    )
    def agg(src_hbm, dst_hbm, table_hbm, out_hbm, *refs):
        isrc = list(refs[0:NI])
        idst = list(refs[NI:2 * NI])
        rows = list(refs[2 * NI:2 * NI + NBUF])
        isem = list(refs[2 * NI + NBUF:3 * NI + NBUF])
        gsem = list(refs[3 * NI + NBUF:3 * NI + 2 * NBUF])
        ssem = list(refs[3 * NI + 2 * NBUF:3 * NI + 3 * NBUF])
        acc = refs[3 * NI + 3 * NBUF]
        c = lax.axis_index("c")
        s = lax.axis_index("s")
        if feature_split:
            src_base = c * E + s * ew
            dst_base = s * ew
        else:
            src_base = (s * NC + c) * ew
            dst_base = src_base

        def iload(j, h):
            pltpu.async_copy(src_hbm.at[pl.ds(src_base + j * K, K)],
                             isrc[h], isem[h])
            pltpu.async_copy(dst_hbm.at[pl.ds(dst_base + j * K, K)],
                             idst[h], isem[h])

        def iload_wait(h):
            pltpu.make_async_copy(src_hbm.at[pl.ds(0, K)], isrc[h],
                                  isem[h]).wait()
            pltpu.make_async_copy(dst_hbm.at[pl.ds(0, K)], idst[h],
                                  isem[h]).wait()

        def gather(h, b):
            pltpu.async_copy(table_hbm.at[isrc[h]], rows[b], gsem[b])

        def gather_wait(h, b):
            pltpu.make_async_copy(table_hbm.at[isrc[h]], rows[b],
                                  gsem[b]).wait()

        def scatter(h, b):
            pltpu.async_copy(rows[b], acc.at[idst[h]], ssem[b], add=True)

        def scatter_wait(h, b):
            pltpu.make_async_copy(rows[b], acc.at[idst[h]], ssem[b]).wait()

        for j in range(LI):
            iload(j, j)
        def zrow(r, carry):
            for jj in range(128 // 16):
                rows[0][r, pl.ds(jj * 16, 16)] = jnp.zeros((16,), _f32)
            return carry
        lax.fori_loop(0, K, zrow, 0)
        for t in range(zr // K):
            pltpu.sync_copy(rows[0], acc.at[pl.ds(s * zr + t * K, K)])
        for j in range(LG):
            iload_wait(j)
            gather(j, j)
        plsc.subcore_barrier()

        def body(i, h):
            b = h % NBUF
            jg = i + LG
            @pl.when(jnp.logical_and(i >= LG, jg < n))
            def _():
                scatter_wait((h - LG) % NI, (b + LG) % NBUF)
            ji = i + LI
            @pl.when(ji < n)
            def _():
                iload(ji, (h + LI) % NI)
            @pl.when(jg < n)
            def _():
                iload_wait((h + LG) % NI)
                gather((h + LG) % NI, (b + LG) % NBUF)
            gather_wait(h, b)
            scatter(h, b)

        def group(g, carry):
            for u in range(NI):
                body(g * NI + u, u)
            return carry
        lax.fori_loop(0, n // NI, group, 0)
        for i in range(NI * (n // NI), n):
            body(jnp.int32(i), i % NI)
        for k in range(NBUF):
            scatter_wait((n - NBUF + k) % NI, (n - NBUF + k) % NBUF)
        plsc.subcore_barrier()
        pltpu.sync_copy(acc.at[pl.ds(s * zr, zr)],
                        out_hbm.at[pl.ds(c * NP + s * zr, zr)])

    return agg


_agg_l1 = _make_agg(feature_split=False)
_agg_l2 = _make_agg(feature_split=True)


_BN = 2000
_NB = N // _BN


def _bn_apply(u, s_scr, q_scr, g_ref, be_ref):
    mu = s_scr[...] * (1.0 / N)
    var = q_scr[...] * (1.0 / N) - mu * mu
    inv = lax.rsqrt(var + 1e-5)
    return jnp.maximum((u - mu) * (inv * g_ref[...]) + be_ref[...], 0.0)


def _mlp_phase(h, wa_ref, ba_ref, wb_ref, bb_ref, i, u_scr, s_scr, q_scr):
    t = jnp.maximum(
        jnp.dot(h, wa_ref[...], preferred_element_type=_f32) + ba_ref[...], 0.0)
    u = jnp.dot(t, wb_ref[...], preferred_element_type=_f32) + bb_ref[...]
    u_scr[pl.ds(i * _BN, _BN), :] = u
    @pl.when(i == 0)
    def _():
        s_scr[...] = jnp.zeros_like(s_scr)
        q_scr[...] = jnp.zeros_like(q_scr)
    s_scr[...] += jnp.sum(u, axis=0, keepdims=True)
    q_scr[...] += jnp.sum(u * u, axis=0, keepdims=True)


def _layer1(x_ref, agg_ref, wa_ref, ba_ref, wb_ref, bb_ref, g_ref, be_ref,
            o_ref, u_scr, s_scr, q_scr):
    p, i = pl.program_id(0), pl.program_id(1)
    @pl.when(p == 0)
    def _():
        h = x_ref[...] + agg_ref[0] + agg_ref[1]
        _mlp_phase(h, wa_ref, ba_ref, wb_ref, bb_ref, i, u_scr, s_scr, q_scr)
    @pl.when(p == 1)
    def _():
        hb = _bn_apply(u_scr[pl.ds(i * _BN, _BN), :], s_scr, q_scr,
                       g_ref, be_ref)
        o_ref[0] = hb[:, :128]
        o_ref[1] = hb[:, 128:]


def _layer2(hb_ref, agg_ref, wa_ref, ba_ref, wb_ref, bb_ref, g_ref, be_ref,
            wfc_ref, bfc_ref, o_ref, u_scr, s_scr, q_scr):
    p, i = pl.program_id(0), pl.program_id(1)
    @pl.when(p == 0)
    def _():
        h = jnp.concatenate(
            [hb_ref[0] + agg_ref[0], hb_ref[1] + agg_ref[1]], axis=1)
        _mlp_phase(h, wa_ref, ba_ref, wb_ref, bb_ref, i, u_scr, s_scr, q_scr)
    @pl.when(p == 1)
    def _():
        hc = _bn_apply(u_scr[pl.ds(i * _BN, _BN), :], s_scr, q_scr,
                       g_ref, be_ref)
        logits = (jnp.dot(hc, wfc_ref[...], preferred_element_type=_f32)
                  + bfc_ref[...])
        m = jnp.max(logits, axis=1, keepdims=True)
        lse = jnp.log(jnp.sum(jnp.exp(logits - m), axis=1, keepdims=True))
        o_ref[...] = logits - m - lse


def _p0_row_spec(w):
    return pl.BlockSpec((_BN, w), lambda p, i: (i * (1 - p), 0))


def _p0_split_spec():
    return pl.BlockSpec((2, _BN, 128), lambda p, i: (0, i * (1 - p), 0))


def _full_spec(shape):
    return pl.BlockSpec(shape, lambda p, i: tuple(0 for _ in shape))


_SCRATCH = [
    pltpu.VMEM((N, H), _f32),
    pltpu.VMEM((1, H), _f32),
    pltpu.VMEM((1, H), _f32),
]


def kernel(x, edge_index, W1a, b1a, W1b, b1b, g1, be1,
           W2a, b2a, W2b, b2b, g2, be2, Wfc, bfc):
    src = edge_index[0]
    dst = edge_index[1]
    src_both = jnp.concatenate([src, src + N], axis=0)

    agg1 = _agg_l1(src, dst, x).reshape(2, NP, 128)[:, :N]
    hb1 = pl.pallas_call(
        _layer1,
        grid=(2, _NB),
        in_specs=[_p0_row_spec(DIN), _p0_split_spec(),
                  _full_spec((DIN, H)), _full_spec((1, H)),
                  _full_spec((H, H)), _full_spec((1, H)),
                  _full_spec((1, H)), _full_spec((1, H))],
        out_specs=pl.BlockSpec((2, _BN, 128), lambda p, i: (0, i * p, 0)),
        out_shape=jax.ShapeDtypeStruct((2, N, 128), _f32),
        scratch_shapes=_SCRATCH,
    )(x, agg1, W1a, b1a.reshape(1, H), W1b, b1b.reshape(1, H),
      g1.reshape(1, H), be1.reshape(1, H))

    agg2 = _agg_l2(src_both, dst, hb1.reshape(2 * N, 128)
                   ).reshape(2, NP, 128)[:, :N]
    out = pl.pallas_call(
        _layer2,
        grid=(2, _NB),
        in_specs=[_p0_split_spec(), _p0_split_spec(),
                  _full_spec((H, H)), _full_spec((1, H)),
                  _full_spec((H, H)), _full_spec((1, H)),
                  _full_spec((1, H)), _full_spec((1, H)),
                  _full_spec((H, C)), _full_spec((1, C))],
        out_specs=pl.BlockSpec((_BN, C), lambda p, i: (i * p, 0)),
        out_shape=jax.ShapeDtypeStruct((N, C), _f32),
        scratch_shapes=_SCRATCH,
    )(hb1, agg2, W2a, b2a.reshape(1, H), W2b, b2b.reshape(1, H),
      g2.reshape(1, H), be2.reshape(1, H), Wfc, bfc.reshape(1, C))
    return out
